# baseline (speedup 1.0000x reference)
import functools

import jax
import jax.numpy as jnp
from jax import lax
from jax.experimental import pallas as pl
from jax.experimental.pallas import tpu as pltpu
from jax.experimental.pallas import tpu_sc as plsc

EMB = 300
NROW = 25
LANES = 16
NPAD = 32
FULL_CHUNKS = 18
TAIL_OFF = EMB - LANES
WPAD = FULL_CHUNKS * LANES + LANES


def _make_sc_call():
    mesh = plsc.VectorSubcoreMesh(core_axis_name="c", subcore_axis_name="s")

    @functools.partial(
        pl.kernel,
        out_type=jax.ShapeDtypeStruct((NPAD,), jnp.float32),
        mesh=mesh,
        compiler_params=pltpu.CompilerParams(
            needs_layout_passes=False, use_tc_tiling_on_sc=True,
            skip_device_barrier=True),
        scratch_types=[
            pltpu.VMEM((NPAD,), jnp.int32),
            pltpu.VMEM((WPAD,), jnp.float32),
            pltpu.VMEM((LANES,), jnp.float32),
            pltpu.VMEM((NPAD, EMB), jnp.float32),
            pltpu.VMEM((NPAD, LANES), jnp.float32),
            pltpu.VMEM((NPAD,), jnp.float32),
            pltpu.SemaphoreType.DMA,
        ],
    )
    def sc_fn(idx_hbm, wp_hbm, b_hbm, table_hbm, out_hbm,
              idx_v, w_v, b_v, rows_v, acc_v, out_v, sem):
        cid = lax.axis_index("c")
        sid = lax.axis_index("s")

        @pl.when(jnp.logical_and(cid == 0, sid == 0))
        def _():
            pltpu.sync_copy(idx_hbm, idx_v)
            pltpu.sync_copy(wp_hbm, w_v)
            pltpu.sync_copy(b_hbm, b_v)
            iv0 = idx_v[pl.ds(0, LANES)]
            iv1 = idx_v[pl.ds(LANES, LANES)]
            copies = []
            for i in range(NROW):
                r = iv0[i] if i < LANES else iv1[i - LANES]
                copies.append(pltpu.async_copy(
                    table_hbm.at[pl.ds(r, 1), :],
                    rows_v.at[pl.ds(i, 1), :], sem))
            for c in copies:
                c.wait()

            wchunks = [w_v[pl.ds(c * LANES, LANES)] for c in range(FULL_CHUNKS)]
            wtail = w_v[pl.ds(FULL_CHUNKS * LANES, LANES)]

            lane = lax.iota(jnp.int32, LANES)
            zeros = jnp.zeros((LANES,), jnp.float32)
            for i in range(NROW, NPAD):
                acc_v[i, pl.ds(0, LANES)] = zeros
            for i in range(NROW):
                acc = rows_v[i, pl.ds(0, LANES)] * wchunks[0]
                for c in range(1, FULL_CHUNKS):
                    acc = acc + rows_v[i, pl.ds(c * LANES, LANES)] * wchunks[c]
                acc = acc + rows_v[i, pl.ds(TAIL_OFF, LANES)] * wtail
                acc_v[i, pl.ds(0, LANES)] = acc

            bias = b_v[...]
            for h in range(NPAD // LANES):
                rows_idx = lane + (h * LANES)
                tot = plsc.load_gather(
                    acc_v, [rows_idx, jnp.zeros((LANES,), jnp.int32)])
                for j in range(1, LANES):
                    tot = tot + plsc.load_gather(
                        acc_v, [rows_idx, jnp.full((LANES,), j, jnp.int32)])
                x = tot + bias
                out_v[pl.ds(h * LANES, LANES)] = 1.0 / (1.0 + jnp.exp(-x))

            pltpu.sync_copy(out_v, out_hbm)

    return sc_fn


_SC_CALL = _make_sc_call()


def kernel(input_words, table, W, b):
    idx = jnp.zeros((NPAD,), jnp.int32).at[:NROW].set(input_words[:, -1])
    w0 = W[:, 0]
    wp = jnp.concatenate(
        [w0[: FULL_CHUNKS * LANES], jnp.zeros((4,), jnp.float32), w0[FULL_CHUNKS * LANES:]]
    )
    bvec = jnp.full((LANES,), b[0], jnp.float32)
    out = _SC_CALL(idx, wp, bvec, table)
    return out[:NROW]


# speedup vs baseline: 1.0121x; 1.0121x over previous
import functools

import jax
import jax.numpy as jnp
from jax import lax
from jax.experimental import pallas as pl
from jax.experimental.pallas import tpu as pltpu
from jax.experimental.pallas import tpu_sc as plsc

EMB = 300
NROW = 25
LANES = 16
NPAD = 32
FULL_CHUNKS = 18
TAIL_OFF = EMB - LANES
WPAD = FULL_CHUNKS * LANES + LANES


def _make_sc_call():
    mesh = plsc.VectorSubcoreMesh(
        core_axis_name="c", subcore_axis_name="s", num_cores=1)

    @functools.partial(
        pl.kernel,
        out_type=jax.ShapeDtypeStruct((NPAD,), jnp.float32),
        mesh=mesh,
        compiler_params=pltpu.CompilerParams(
            needs_layout_passes=False, use_tc_tiling_on_sc=True,
            skip_device_barrier=True),
        scratch_types=[
            pltpu.VMEM((NPAD,), jnp.int32),
            pltpu.VMEM((WPAD,), jnp.float32),
            pltpu.VMEM((LANES,), jnp.float32),
            pltpu.VMEM((NPAD, EMB), jnp.float32),
            pltpu.VMEM((NPAD, LANES), jnp.float32),
            pltpu.VMEM((NPAD,), jnp.float32),
            pltpu.SemaphoreType.DMA,
        ],
    )
    def sc_fn(idx_hbm, wp_hbm, b_hbm, table_hbm, out_hbm,
              idx_v, w_v, b_v, rows_v, acc_v, out_v, sem):
        cid = lax.axis_index("c")
        sid = lax.axis_index("s")

        @pl.when(jnp.logical_and(cid == 0, sid == 0))
        def _():
            pltpu.sync_copy(idx_hbm, idx_v)
            pltpu.sync_copy(wp_hbm, w_v)
            pltpu.sync_copy(b_hbm, b_v)
            iv0 = idx_v[pl.ds(0, LANES)]
            iv1 = idx_v[pl.ds(LANES, LANES)]
            copies = []
            for i in range(NROW):
                r = iv0[i] if i < LANES else iv1[i - LANES]
                copies.append(pltpu.async_copy(
                    table_hbm.at[pl.ds(r, 1), :],
                    rows_v.at[pl.ds(i, 1), :], sem))
            for c in copies:
                c.wait()

            wchunks = [w_v[pl.ds(c * LANES, LANES)] for c in range(FULL_CHUNKS)]
            wtail = w_v[pl.ds(FULL_CHUNKS * LANES, LANES)]

            lane = lax.iota(jnp.int32, LANES)
            zeros = jnp.zeros((LANES,), jnp.float32)
            for i in range(NROW, NPAD):
                acc_v[i, pl.ds(0, LANES)] = zeros
            for i in range(NROW):
                acc = rows_v[i, pl.ds(0, LANES)] * wchunks[0]
                for c in range(1, FULL_CHUNKS):
                    acc = acc + rows_v[i, pl.ds(c * LANES, LANES)] * wchunks[c]
                acc = acc + rows_v[i, pl.ds(TAIL_OFF, LANES)] * wtail
                acc_v[i, pl.ds(0, LANES)] = acc

            bias = b_v[...]
            for h in range(NPAD // LANES):
                rows_idx = lane + (h * LANES)
                tot = plsc.load_gather(
                    acc_v, [rows_idx, jnp.zeros((LANES,), jnp.int32)])
                for j in range(1, LANES):
                    tot = tot + plsc.load_gather(
                        acc_v, [rows_idx, jnp.full((LANES,), j, jnp.int32)])
                x = tot + bias
                out_v[pl.ds(h * LANES, LANES)] = 1.0 / (1.0 + jnp.exp(-x))

            pltpu.sync_copy(out_v, out_hbm)

    return sc_fn


_SC_CALL = _make_sc_call()


def kernel(input_words, table, W, b):
    idx = jnp.zeros((NPAD,), jnp.int32).at[:NROW].set(input_words[:, -1])
    w0 = W[:, 0]
    wp = jnp.concatenate(
        [w0[: FULL_CHUNKS * LANES], jnp.zeros((4,), jnp.float32), w0[FULL_CHUNKS * LANES:]]
    )
    bvec = jnp.full((LANES,), b[0], jnp.float32)
    out = _SC_CALL(idx, wp, bvec, table)
    return out[:NROW]


# speedup vs baseline: 2.0486x; 2.0242x over previous
import functools

import jax
import jax.numpy as jnp
from jax import lax
from jax.experimental import pallas as pl
from jax.experimental.pallas import tpu as pltpu
from jax.experimental.pallas import tpu_sc as plsc

EMB = 300
NROW = 25
LANES = 16
NPAD = 32
FULL_CHUNKS = 18
TAIL_OFF = EMB - LANES
WPAD = FULL_CHUNKS * LANES + LANES
BLK = 128


def _make_sc_call():
    mesh = plsc.VectorSubcoreMesh(
        core_axis_name="c", subcore_axis_name="s", num_cores=1)

    @functools.partial(
        pl.kernel,
        out_type=jax.ShapeDtypeStruct((NPAD,), jnp.float32),
        mesh=mesh,
        compiler_params=pltpu.CompilerParams(
            needs_layout_passes=False, use_tc_tiling_on_sc=True,
            skip_device_barrier=True),
        scratch_types=[
            pltpu.VMEM((NPAD,), jnp.int32),
            pltpu.VMEM((WPAD,), jnp.float32),
            pltpu.VMEM((LANES,), jnp.float32),
            pltpu.VMEM((EMB, BLK), jnp.float32),
            pltpu.VMEM((EMB, BLK), jnp.float32),
            pltpu.VMEM((NPAD, LANES), jnp.float32),
            pltpu.VMEM((NPAD,), jnp.float32),
            pltpu.SemaphoreType.DMA,
            pltpu.SemaphoreType.DMA,
        ],
    )
    def sc_fn(idx_hbm, wp_hbm, b_hbm, tableT_hbm, out_hbm,
              idx_v, w_v, b_v, blk0_v, blk1_v, acc_v, out_v, sem0, sem1):
        cid = lax.axis_index("c")
        sid = lax.axis_index("s")

        @pl.when(jnp.logical_and(cid == 0, sid == 0))
        def _():
            pltpu.sync_copy(idx_hbm, idx_v)
            pltpu.sync_copy(wp_hbm, w_v)
            pltpu.sync_copy(b_hbm, b_v)
            iv0 = idx_v[pl.ds(0, LANES)]
            iv1 = idx_v[pl.ds(LANES, LANES)]

            blks = (blk0_v, blk1_v)
            sems = (sem0, sem1)

            def row_idx(i):
                return iv0[i] if i < LANES else iv1[i - LANES]

            def fire(i):
                r = row_idx(i)
                tb = pl.multiple_of((r // BLK) * BLK, BLK)
                cp = pltpu.async_copy(
                    tableT_hbm.at[:, pl.ds(tb, BLK)], blks[i % 2],
                    sems[i % 2])
                return cp, r - tb

            wchunks = [w_v[pl.ds(c * LANES, LANES)] for c in range(FULL_CHUNKS)]
            wtail = w_v[pl.ds(FULL_CHUNKS * LANES, LANES)]
            lane = lax.iota(jnp.int32, LANES)
            zeros = jnp.zeros((LANES,), jnp.float32)
            for i in range(NROW, NPAD):
                acc_v[i, pl.ds(0, LANES)] = zeros

            pending = fire(0)
            for i in range(NROW):
                cp, col = pending
                cp.wait()
                if i + 1 < NROW:
                    pending = fire(i + 1)
                blk = blks[i % 2]
                colv = jnp.full((LANES,), col, jnp.int32)
                acc = plsc.load_gather(blk, [lane, colv]) * wchunks[0]
                for c in range(1, FULL_CHUNKS):
                    acc = acc + plsc.load_gather(
                        blk, [c * LANES + lane, colv]) * wchunks[c]
                acc = acc + plsc.load_gather(
                    blk, [TAIL_OFF + lane, colv]) * wtail
                acc_v[i, pl.ds(0, LANES)] = acc

            bias = b_v[...]
            for h in range(NPAD // LANES):
                rows_idx = lane + (h * LANES)
                tot = plsc.load_gather(
                    acc_v, [rows_idx, jnp.zeros((LANES,), jnp.int32)])
                for j in range(1, LANES):
                    tot = tot + plsc.load_gather(
                        acc_v, [rows_idx, jnp.full((LANES,), j, jnp.int32)])
                x = tot + bias
                out_v[pl.ds(h * LANES, LANES)] = 1.0 / (1.0 + jnp.exp(-x))

            pltpu.sync_copy(out_v, out_hbm)

    return sc_fn


_SC_CALL = _make_sc_call()


def kernel(input_words, table, W, b):
    idx = jnp.zeros((NPAD,), jnp.int32).at[:NROW].set(input_words[:, -1])
    w0 = W[:, 0]
    wp = jnp.concatenate(
        [w0[: FULL_CHUNKS * LANES], jnp.zeros((4,), jnp.float32), w0[FULL_CHUNKS * LANES:]]
    )
    bvec = jnp.full((LANES,), b[0], jnp.float32)
    out = _SC_CALL(idx, wp, bvec, jnp.swapaxes(table, 0, 1))
    return out[:NROW]


# speedup vs baseline: 5.6962x; 2.7805x over previous
import functools

import jax
import jax.numpy as jnp
from jax import lax
from jax.experimental import pallas as pl
from jax.experimental.pallas import tpu as pltpu
from jax.experimental.pallas import tpu_sc as plsc

EMB = 300
NROW = 25
LANES = 16
NPAD = 32
FULL_CHUNKS = 18
TAIL_OFF = EMB - LANES
WPAD = FULL_CHUNKS * LANES + LANES
BLK = 128
NSECOND = NROW - LANES


def _make_sc_call():
    mesh = plsc.VectorSubcoreMesh(
        core_axis_name="c", subcore_axis_name="s", num_cores=1)

    @functools.partial(
        pl.kernel,
        out_type=jax.ShapeDtypeStruct((NPAD,), jnp.float32),
        mesh=mesh,
        compiler_params=pltpu.CompilerParams(
            needs_layout_passes=False, use_tc_tiling_on_sc=True,
            skip_device_barrier=True),
        scratch_types=[
            pltpu.VMEM((NPAD,), jnp.int32),
            pltpu.VMEM((WPAD,), jnp.float32),
            pltpu.VMEM((LANES,), jnp.float32),
            pltpu.VMEM((EMB, BLK), jnp.float32),
            pltpu.VMEM((EMB, BLK), jnp.float32),
            pltpu.VMEM((2 * LANES,), jnp.float32),
            pltpu.VMEM((LANES * 2 * LANES,), jnp.float32),
            pltpu.VMEM((NPAD,), jnp.float32),
            pltpu.VMEM_SHARED((LANES * 2 * LANES,), jnp.float32),
            pltpu.SemaphoreType.DMA,
            pltpu.SemaphoreType.DMA,
        ],
    )
    def sc_fn(idx_hbm, wp_hbm, b_hbm, tableT_hbm, out_hbm,
              idx_v, w_v, b_v, blk0_v, blk1_v, stage_v, flat_v, out_v,
              acc_sh, sem0, sem1):
        sid = lax.axis_index("s")

        pltpu.sync_copy(idx_hbm, idx_v)
        pltpu.sync_copy(wp_hbm, w_v)

        lane = lax.iota(jnp.int32, LANES)
        sidv = jnp.full((LANES,), sid, jnp.int32)
        r0 = plsc.load_gather(idx_v, [sidv])[0]
        r1 = plsc.load_gather(idx_v, [sidv + LANES])[0]

        def fire(r, blk, sem):
            tb = pl.multiple_of((r // BLK) * BLK, BLK)
            cp = pltpu.async_copy(
                tableT_hbm.at[:, pl.ds(tb, BLK)], blk, sem)
            return cp, r - tb

        cp0, col0 = fire(r0, blk0_v, sem0)
        has2 = sid < NSECOND
        cp1, col1 = fire(jnp.where(has2, r1, r0), blk1_v, sem1)

        wchunks = [w_v[pl.ds(c * LANES, LANES)] for c in range(FULL_CHUNKS)]
        wtail = w_v[pl.ds(FULL_CHUNKS * LANES, LANES)]

        def dot_column(blk, col):
            colv = jnp.full((LANES,), col, jnp.int32)
            acc = plsc.load_gather(blk, [lane, colv]) * wchunks[0]
            for c in range(1, FULL_CHUNKS):
                acc = acc + plsc.load_gather(
                    blk, [c * LANES + lane, colv]) * wchunks[c]
            return acc + plsc.load_gather(
                blk, [TAIL_OFF + lane, colv]) * wtail

        cp0.wait()
        stage_v[pl.ds(0, LANES)] = dot_column(blk0_v, col0)
        cp1.wait()
        acc1 = dot_column(blk1_v, col1)
        stage_v[pl.ds(LANES, LANES)] = jnp.where(
            has2, acc1, jnp.zeros((LANES,), jnp.float32))

        off = pl.multiple_of(sid * (2 * LANES), 2 * LANES)
        pltpu.sync_copy(stage_v, acc_sh.at[pl.ds(off, 2 * LANES)])
        plsc.subcore_barrier()

        @pl.when(sid == 0)
        def _():
            pltpu.sync_copy(b_hbm, b_v)
            pltpu.sync_copy(acc_sh, flat_v)
            bias = b_v[...]
            for h in range(NPAD // LANES):
                base = lane * (2 * LANES) + h * LANES
                tot = plsc.load_gather(flat_v, [base])
                for j in range(1, LANES):
                    tot = tot + plsc.load_gather(flat_v, [base + j])
                x = tot + bias
                out_v[pl.ds(h * LANES, LANES)] = 1.0 / (1.0 + jnp.exp(-x))
            pltpu.sync_copy(out_v, out_hbm)

    return sc_fn


_SC_CALL = _make_sc_call()


def kernel(input_words, table, W, b):
    idx = jnp.zeros((NPAD,), jnp.int32).at[:NROW].set(input_words[:, -1])
    w0 = W[:, 0]
    wp = jnp.concatenate(
        [w0[: FULL_CHUNKS * LANES], jnp.zeros((4,), jnp.float32), w0[FULL_CHUNKS * LANES:]]
    )
    bvec = jnp.full((LANES,), b[0], jnp.float32)
    out = _SC_CALL(idx, wp, bvec, jnp.swapaxes(table, 0, 1))
    return out[:NROW]
